# baseline (speedup 1.0000x reference)
import functools

import jax
import jax.numpy as jnp
from jax import lax
from jax.experimental import pallas as pl
from jax.experimental.pallas import tpu as pltpu
from jax.experimental.pallas import tpu_sc as plsc

EPS = 1e-5
COS_EPS = 1e-8
D = 128
OUT = 64


_ROW_BLK = 1000


def _atan_pos(n):
    inv = n > 1.0
    t = jnp.where(inv, 1.0 / jnp.maximum(n, 1e-30), n)
    t = t / (1.0 + jnp.sqrt(1.0 + t * t))
    t = t / (1.0 + jnp.sqrt(1.0 + t * t))
    z2 = t * t
    p = t * (1.0 + z2 * (-1.0 / 3.0 + z2 * (1.0 / 5.0 + z2 * (-1.0 / 7.0))))
    p = 4.0 * p
    return jnp.where(inv, (jnp.pi / 2.0) - p, p)


def _head_body(xE_ref, xH_ref, xS_ref, W_ref, b_ref, y_ref):
    xE = xE_ref[...]
    xH = xH_ref[...]
    xS = xS_ref[...]

    nh = jnp.sqrt(jnp.sum(xH * xH, axis=-1))
    nh_c = jnp.clip(nh, EPS, 1.0 - EPS)
    scale_h = (0.5 * jnp.log((1.0 + nh_c) / (1.0 - nh_c))
               / jnp.maximum(nh, EPS))

    ns = jnp.sqrt(jnp.sum(xS * xS, axis=-1))
    ns_safe = jnp.maximum(ns, EPS)
    scale_s = _atan_pos(ns_safe) / ns_safe

    W = W_ref[...]
    dn = (((1,), (1,)), ((), ()))
    acc = lax.dot_general(xE, W[:, 0:D], dn, preferred_element_type=jnp.float32)
    acc += scale_h[:, None] * lax.dot_general(
        xH, W[:, D:2 * D], dn, preferred_element_type=jnp.float32)
    acc += scale_s[:, None] * lax.dot_general(
        xS, W[:, 2 * D:3 * D], dn, preferred_element_type=jnp.float32)
    acc += b_ref[...]

    norm = jnp.sqrt(jnp.sum(acc * acc, axis=-1))
    y = acc * (1.0 / jnp.maximum(norm, COS_EPS))[:, None]
    y_ref[...] = y.astype(jnp.bfloat16)


def _dense_head(x_E, x_H, x_S, W, b):
    n = x_E.shape[0]
    grid = n // _ROW_BLK
    return pl.pallas_call(
        _head_body,
        grid=(grid,),
        in_specs=[
            pl.BlockSpec((_ROW_BLK, D), lambda i: (i, 0)),
            pl.BlockSpec((_ROW_BLK, D), lambda i: (i, 0)),
            pl.BlockSpec((_ROW_BLK, D), lambda i: (i, 0)),
            pl.BlockSpec((OUT, 3 * D), lambda i: (0, 0)),
            pl.BlockSpec((1, OUT), lambda i: (0, 0)),
        ],
        out_specs=pl.BlockSpec((_ROW_BLK, OUT), lambda i: (i, 0)),
        out_shape=jax.ShapeDtypeStruct((n, OUT), jnp.bfloat16),
    )(x_E, x_H, x_S, W, b.reshape(1, OUT))


_NC = 2
_NS = 16
_NW = _NC * _NS
_B = 400
_L = 16


_BITREV = (0, 8, 4, 12, 2, 10, 6, 14, 1, 9, 5, 13, 3, 11, 7, 15)

_GDN = lax.GatherDimensionNumbers(
    offset_dims=(), collapsed_slice_dims=(0,), start_index_map=(0,))


def _rot(v, idx):
    return lax.gather(v, idx[:, None], _GDN, slice_sizes=(1,),
                      mode=lax.GatherScatterMode.PROMISE_IN_BOUNDS)


def _make_edge_dot(e_lab):
    e_per_w = e_lab // (_NW // 2)
    n_chunk = e_per_w // _B
    mesh = plsc.VectorSubcoreMesh(core_axis_name="c", subcore_axis_name="s")

    _NBUF = 5
    _K = 25

    @functools.partial(
        pl.kernel,
        mesh=mesh,
        out_type=jax.ShapeDtypeStruct((2 * e_lab,), jnp.float32),
        compiler_params=pltpu.CompilerParams(use_tc_tiling_on_sc=False,
                                             needs_layout_passes=False),
        scratch_types=[
            pltpu.VMEM((_K * _B,), jnp.int32),
            pltpu.VMEM((_K * _B,), jnp.int32),
            [pltpu.VMEM((_B, OUT), jnp.bfloat16)] * _NBUF,
            pltpu.VMEM((_K * _B,), jnp.float32),
            [pltpu.SemaphoreType.DMA] * _NBUF,
            [pltpu.SemaphoreType.DMA] * _NBUF,
        ],
    )
    def edge_dot(y_hbm, pos_hbm, neg_hbm, out_hbm,
                 src_blk, dst_blk, rows, sco_blk, semA, semB):
        wid = lax.axis_index("s") * _NC + lax.axis_index("c")
        lanes = lax.iota(jnp.int32, _L)
        rot_idx = {sh: (lanes + sh) % _L for sh in (8, 4, 2, 1)}
        rot_idx_neg = {sh: (lanes - sh) % _L for sh in (8, 4, 2, 1)}
        masks = {sh: (lanes & sh) == 0 for sh in (8, 4, 2, 1)}

        def run_slice(eli_hbm, in_base, out_base):
            def issue1(jj, b):
                pltpu.async_copy(
                    y_hbm.at[src_blk.at[pl.ds(jj * _B, _B)]],
                    rows[b], semA[b])

            def issue2(jj, b):
                pltpu.make_async_copy(
                    y_hbm.at[src_blk.at[pl.ds(jj * _B, _B)]],
                    rows[b], semA[b]).wait()
                pltpu.async_copy(
                    y_hbm.at[dst_blk.at[pl.ds(jj * _B, _B)]],
                    rows[b], semB[b], add=True)

            def compute(jj, b):
                sr = rows[b]
                hi_mask = jnp.full((_L,), 0xFFFF0000, jnp.uint32)

                def grp_body(g, c2):
                    cur = []
                    for s in range(_L):
                        e = g * _L + _BITREV[s]
                        p = None
                        for k in range(OUT // 32):
                            u = plsc.bitcast(sr[e, pl.ds(k * 32, 32)],
                                             jnp.uint32)
                            lo = plsc.bitcast(u << 16, jnp.float32)
                            hi = plsc.bitcast(u & hi_mask, jnp.float32)
                            q = lo * lo + hi * hi
                            p = q if p is None else p + q
                        cur.append(p)
                    for sh in (8, 4, 2, 1):
                        nxt = []
                        for i in range(len(cur) // 2):
                            u, w = cur[2 * i], cur[2 * i + 1]
                            nxt.append(jnp.where(
                                masks[sh],
                                u + _rot(u, rot_idx[sh]),
                                w + _rot(w, rot_idx_neg[sh])))
                        cur = nxt
                    sco_blk[pl.ds(jj * _B + g * _L, _L)] = 0.5 * cur[0] - 1.0
                    return c2

                lax.fori_loop(0, _B // _L, grp_body, 0)

            def finish(jj, b):
                pltpu.make_async_copy(
                    y_hbm.at[dst_blk.at[pl.ds(jj * _B, _B)]],
                    rows[b], semB[b]).wait()
                compute(jj, b)

            def block_body(bi, carry):
                boff = in_base + bi * _K * _B
                pltpu.sync_copy(eli_hbm.at[0, pl.ds(boff, _K * _B)], src_blk)
                pltpu.sync_copy(eli_hbm.at[1, pl.ds(boff, _K * _B)], dst_blk)
                for p in range(_NBUF):
                    issue1(p, p)
                issue2(0, 0)
                issue2(1, 1)

                def inner(h, c2):
                    for u in range(_NBUF):
                        jj = h * _NBUF + u
                        finish(jj, u)

                        @pl.when(jj + _NBUF < _K)
                        def _():
                            issue1(jj + _NBUF, u)

                        @pl.when(jj + 2 < _K)
                        def _():
                            issue2(jj + 2, (u + 2) % _NBUF)
                    return c2

                lax.fori_loop(0, _K // _NBUF, inner, 0)
                for jj in range((_K // _NBUF) * _NBUF, _K):
                    finish(jj, jj % _NBUF)
                pltpu.sync_copy(
                    sco_blk,
                    out_hbm.at[pl.ds(out_base + bi * _K * _B, _K * _B)])
                return carry

            lax.fori_loop(0, n_chunk // _K, block_body, 0)

        half_base = (wid % (_NW // 2)) * e_per_w

        @pl.when(wid < _NW // 2)
        def _():
            run_slice(pos_hbm, half_base, half_base)

        @pl.when(wid >= _NW // 2)
        def _():
            run_slice(neg_hbm, half_base, e_lab + half_base)

    return edge_dot


def kernel(x_E, x_H, x_S, edge_label_index, edge_label, neg_edge_index, W, b):
    y = _dense_head(x_E, x_H, x_S, W, b)

    e_lab = edge_label_index.shape[1]
    score = _make_edge_dot(e_lab)(y, edge_label_index, neg_edge_index)

    el = jnp.concatenate(
        [edge_label, jnp.zeros((neg_edge_index.shape[1],), edge_label.dtype)])
    return (score, el)


# speedup vs baseline: 1.0063x; 1.0063x over previous
import functools

import jax
import jax.numpy as jnp
from jax import lax
from jax.experimental import pallas as pl
from jax.experimental.pallas import tpu as pltpu
from jax.experimental.pallas import tpu_sc as plsc

EPS = 1e-5
COS_EPS = 1e-8
D = 128
OUT = 64


_ROW_BLK = 1000


def _atan_pos(n):
    inv = n > 1.0
    t = jnp.where(inv, 1.0 / jnp.maximum(n, 1e-30), n)
    t = t / (1.0 + jnp.sqrt(1.0 + t * t))
    t = t / (1.0 + jnp.sqrt(1.0 + t * t))
    z2 = t * t
    p = t * (1.0 + z2 * (-1.0 / 3.0 + z2 * (1.0 / 5.0 + z2 * (-1.0 / 7.0))))
    p = 4.0 * p
    return jnp.where(inv, (jnp.pi / 2.0) - p, p)


def _head_body(xE_ref, xH_ref, xS_ref, W_ref, b_ref, y_ref):
    xE = xE_ref[...]
    xH = xH_ref[...]
    xS = xS_ref[...]

    nh = jnp.sqrt(jnp.sum(xH * xH, axis=-1))
    nh_c = jnp.clip(nh, EPS, 1.0 - EPS)
    scale_h = (0.5 * jnp.log((1.0 + nh_c) / (1.0 - nh_c))
               / jnp.maximum(nh, EPS))

    ns = jnp.sqrt(jnp.sum(xS * xS, axis=-1))
    ns_safe = jnp.maximum(ns, EPS)
    scale_s = _atan_pos(ns_safe) / ns_safe

    W = W_ref[...]
    dn = (((1,), (1,)), ((), ()))
    acc = lax.dot_general(xE, W[:, 0:D], dn, preferred_element_type=jnp.float32)
    acc += scale_h[:, None] * lax.dot_general(
        xH, W[:, D:2 * D], dn, preferred_element_type=jnp.float32)
    acc += scale_s[:, None] * lax.dot_general(
        xS, W[:, 2 * D:3 * D], dn, preferred_element_type=jnp.float32)
    acc += b_ref[...]

    norm = jnp.sqrt(jnp.sum(acc * acc, axis=-1))
    y = acc * (1.0 / jnp.maximum(norm, COS_EPS))[:, None]
    y_ref[...] = y.astype(jnp.bfloat16)


def _dense_head(x_E, x_H, x_S, W, b):
    n = x_E.shape[0]
    grid = n // _ROW_BLK
    return pl.pallas_call(
        _head_body,
        grid=(grid,),
        in_specs=[
            pl.BlockSpec((_ROW_BLK, D), lambda i: (i, 0)),
            pl.BlockSpec((_ROW_BLK, D), lambda i: (i, 0)),
            pl.BlockSpec((_ROW_BLK, D), lambda i: (i, 0)),
            pl.BlockSpec((OUT, 3 * D), lambda i: (0, 0)),
            pl.BlockSpec((1, OUT), lambda i: (0, 0)),
        ],
        out_specs=pl.BlockSpec((_ROW_BLK, OUT), lambda i: (i, 0)),
        out_shape=jax.ShapeDtypeStruct((n, OUT), jnp.bfloat16),
    )(x_E, x_H, x_S, W, b.reshape(1, OUT))


_NC = 2
_NS = 16
_NW = _NC * _NS
_B = 400
_L = 16


_BITREV = (0, 8, 4, 12, 2, 10, 6, 14, 1, 9, 5, 13, 3, 11, 7, 15)

_GDN = lax.GatherDimensionNumbers(
    offset_dims=(), collapsed_slice_dims=(0,), start_index_map=(0,))


def _rot(v, idx):
    return lax.gather(v, idx[:, None], _GDN, slice_sizes=(1,),
                      mode=lax.GatherScatterMode.PROMISE_IN_BOUNDS)


def _make_edge_dot(e_lab):
    e_per_w = e_lab // (_NW // 2)
    n_chunk = e_per_w // _B
    mesh = plsc.VectorSubcoreMesh(core_axis_name="c", subcore_axis_name="s")

    _NBUF = 3
    _K = 25

    @functools.partial(
        pl.kernel,
        mesh=mesh,
        out_type=jax.ShapeDtypeStruct((2 * e_lab,), jnp.float32),
        compiler_params=pltpu.CompilerParams(use_tc_tiling_on_sc=False,
                                             needs_layout_passes=False),
        scratch_types=[
            pltpu.VMEM((_K * _B,), jnp.int32),
            pltpu.VMEM((_K * _B,), jnp.int32),
            [pltpu.VMEM((_B, OUT), jnp.bfloat16)] * _NBUF,
            pltpu.VMEM((_K * _B,), jnp.float32),
            [pltpu.SemaphoreType.DMA] * _NBUF,
            [pltpu.SemaphoreType.DMA] * _NBUF,
        ],
    )
    def edge_dot(y_hbm, pos_hbm, neg_hbm, out_hbm,
                 src_blk, dst_blk, rows, sco_blk, semA, semB):
        wid = lax.axis_index("s") * _NC + lax.axis_index("c")
        lanes = lax.iota(jnp.int32, _L)
        rot_idx = {sh: (lanes + sh) % _L for sh in (8, 4, 2, 1)}
        rot_idx_neg = {sh: (lanes - sh) % _L for sh in (8, 4, 2, 1)}
        masks = {sh: (lanes & sh) == 0 for sh in (8, 4, 2, 1)}

        def run_slice(eli_hbm, in_base, out_base):
            def issue1(jj, b):
                pltpu.async_copy(
                    y_hbm.at[src_blk.at[pl.ds(jj * _B, _B)]],
                    rows[b], semA[b])

            def issue2(jj, b):
                pltpu.make_async_copy(
                    y_hbm.at[src_blk.at[pl.ds(jj * _B, _B)]],
                    rows[b], semA[b]).wait()
                pltpu.async_copy(
                    y_hbm.at[dst_blk.at[pl.ds(jj * _B, _B)]],
                    rows[b], semB[b], add=True)

            def compute(jj, b):
                sr = rows[b]
                hi_mask = jnp.full((_L,), 0xFFFF0000, jnp.uint32)

                def grp_body(g, c2):
                    cur = []
                    for s in range(_L):
                        e = g * _L + _BITREV[s]
                        p = None
                        for k in range(OUT // 32):
                            u = plsc.bitcast(sr[e, pl.ds(k * 32, 32)],
                                             jnp.uint32)
                            lo = plsc.bitcast(u << 16, jnp.float32)
                            hi = plsc.bitcast(u & hi_mask, jnp.float32)
                            q = lo * lo + hi * hi
                            p = q if p is None else p + q
                        cur.append(p)
                    for sh in (8, 4, 2, 1):
                        nxt = []
                        for i in range(len(cur) // 2):
                            u, w = cur[2 * i], cur[2 * i + 1]
                            nxt.append(jnp.where(
                                masks[sh],
                                u + _rot(u, rot_idx[sh]),
                                w + _rot(w, rot_idx_neg[sh])))
                        cur = nxt
                    sco_blk[pl.ds(jj * _B + g * _L, _L)] = 0.5 * cur[0] - 1.0
                    return c2

                lax.fori_loop(0, _B // _L, grp_body, 0)

            def finish(jj, b):
                pltpu.make_async_copy(
                    y_hbm.at[dst_blk.at[pl.ds(jj * _B, _B)]],
                    rows[b], semB[b]).wait()
                compute(jj, b)

            def block_body(bi, carry):
                boff = in_base + bi * _K * _B
                pltpu.sync_copy(eli_hbm.at[0, pl.ds(boff, _K * _B)], src_blk)
                pltpu.sync_copy(eli_hbm.at[1, pl.ds(boff, _K * _B)], dst_blk)
                issue1(0, 0)
                issue1(1, 1)
                issue2(0, 0)
                issue1(2, 2)
                issue2(1, 1)

                def inner(h, c2):
                    for u in range(_NBUF):
                        jj = h * _NBUF + u
                        finish(jj, u)

                        @pl.when(jj + _NBUF < _K)
                        def _():
                            issue1(jj + _NBUF, u)

                        @pl.when(jj + 2 < _K)
                        def _():
                            issue2(jj + 2, (u + 2) % _NBUF)
                    return c2

                lax.fori_loop(0, _K // _NBUF, inner, 0)
                for jj in range((_K // _NBUF) * _NBUF, _K):
                    finish(jj, jj % _NBUF)
                pltpu.sync_copy(
                    sco_blk,
                    out_hbm.at[pl.ds(out_base + bi * _K * _B, _K * _B)])
                return carry

            lax.fori_loop(0, n_chunk // _K, block_body, 0)

        half_base = (wid % (_NW // 2)) * e_per_w

        @pl.when(wid < _NW // 2)
        def _():
            run_slice(pos_hbm, half_base, half_base)

        @pl.when(wid >= _NW // 2)
        def _():
            run_slice(neg_hbm, half_base, e_lab + half_base)

    return edge_dot


def kernel(x_E, x_H, x_S, edge_label_index, edge_label, neg_edge_index, W, b):
    y = _dense_head(x_E, x_H, x_S, W, b)

    e_lab = edge_label_index.shape[1]
    score = _make_edge_dot(e_lab)(y, edge_label_index, neg_edge_index)

    el = jnp.concatenate(
        [edge_label, jnp.zeros((neg_edge_index.shape[1],), edge_label.dtype)])
    return (score, el)
